# Initial kernel scaffold; baseline (speedup 1.0000x reference)
import jax
import jax.numpy as jnp
from jax.experimental import pallas as pl


def kernel(x, edge_index, W1_rel, b1, W1_root, W2_rel, b2, W2_root):
    raise NotImplementedError("write your pallas kernel here")


# speedup vs baseline: 10.6553x; 10.6553x over previous
import functools

import jax
import jax.numpy as jnp
from jax import lax
from jax.experimental import pallas as pl
from jax.experimental.pallas import tpu as pltpu
from jax.experimental.pallas import tpu_sc as plsc

N = 10000
D_IN = 128
D = 16
E = 320000
NC = 2
NS = 16
NW = NC * NS
E_PER_W = E // NW
CHUNK = 80
NCHUNK = E_PER_W // CHUNK
NP = 10240
ROWS_PER_TILE = NP // NS
ZROWS = 128


def _segsum16(y, src3, dst3):
    mesh = plsc.VectorSubcoreMesh(core_axis_name="c", subcore_axis_name="s")

    @functools.partial(
        pl.kernel,
        out_type=jax.ShapeDtypeStruct((NC, NP, D), jnp.float32),
        mesh=mesh,
        compiler_params=pltpu.CompilerParams(use_tc_tiling_on_sc=False),
        scratch_types=[
            pltpu.VMEM((NCHUNK, CHUNK), jnp.int32),
            pltpu.VMEM((NCHUNK, CHUNK), jnp.int32),
            pltpu.VMEM((CHUNK, D), jnp.float32),
            pltpu.VMEM((ZROWS, D), jnp.float32),
            pltpu.VMEM_SHARED((NP, D), jnp.float32),
        ],
    )
    def k(y_hbm, src_hbm, dst_hbm, out_hbm, src_v, dst_v, rows_v, zbuf, acc):
        cid = lax.axis_index("c")
        sid = lax.axis_index("s")
        wid = cid * NS + sid

        zero16 = jnp.zeros((D,), jnp.float32)

        def zfill(i, _):
            zbuf[i] = zero16
            return 0

        lax.fori_loop(0, ZROWS, zfill, 0)

        def zcopy(i, _):
            pltpu.sync_copy(zbuf, acc.at[pl.ds(sid * ROWS_PER_TILE + i * ZROWS, ZROWS)])
            return 0

        lax.fori_loop(0, ROWS_PER_TILE // ZROWS, zcopy, 0)

        pltpu.sync_copy(src_hbm.at[wid], src_v)
        pltpu.sync_copy(dst_hbm.at[wid], dst_v)
        plsc.subcore_barrier()

        def body(j, _):
            pltpu.sync_copy(y_hbm.at[src_v.at[j]], rows_v)
            pltpu.sync_copy(rows_v, acc.at[dst_v.at[j]], add=True)
            return 0

        lax.fori_loop(0, NCHUNK, body, 0)
        plsc.subcore_barrier()

        pltpu.sync_copy(
            acc.at[pl.ds(sid * ROWS_PER_TILE, ROWS_PER_TILE)],
            out_hbm.at[cid, pl.ds(sid * ROWS_PER_TILE, ROWS_PER_TILE)],
        )

    return k(y, src3, dst3)[:, :N]


_BR = 1000


def _tc_in(x, Wc):
    def body(x_ref, w_ref, y_ref, b_ref):
        z = jnp.dot(x_ref[...], w_ref[...], preferred_element_type=jnp.float32)
        y_ref[...] = z[:, :D]
        b_ref[...] = z[:, D:]

    return pl.pallas_call(
        body,
        grid=(N // _BR,),
        in_specs=[
            pl.BlockSpec((_BR, D_IN), lambda i: (i, 0)),
            pl.BlockSpec((D_IN, 2 * D), lambda i: (0, 0)),
        ],
        out_specs=[
            pl.BlockSpec((_BR, D), lambda i: (i, 0)),
            pl.BlockSpec((_BR, D), lambda i: (i, 0)),
        ],
        out_shape=[
            jax.ShapeDtypeStruct((N, D), jnp.float32),
            jax.ShapeDtypeStruct((N, D), jnp.float32),
        ],
    )(x, Wc)


def _tc_mid(P, base1, b1r, w2relr, w2rootr):
    def body(p_ref, base_ref, b1_ref, wr_ref, wo_ref, y_ref, o_ref):
        h = jnp.maximum(p_ref[0] + p_ref[1] + base_ref[...] + b1_ref[...], 0.0)
        y_ref[...] = h * wr_ref[...]
        o_ref[...] = h * wo_ref[...]

    return pl.pallas_call(
        body,
        grid=(N // _BR,),
        in_specs=[
            pl.BlockSpec((NC, _BR, D), lambda i: (0, i, 0)),
            pl.BlockSpec((_BR, D), lambda i: (i, 0)),
            pl.BlockSpec((1, D), lambda i: (0, 0)),
            pl.BlockSpec((1, D), lambda i: (0, 0)),
            pl.BlockSpec((1, D), lambda i: (0, 0)),
        ],
        out_specs=[
            pl.BlockSpec((_BR, D), lambda i: (i, 0)),
            pl.BlockSpec((_BR, D), lambda i: (i, 0)),
        ],
        out_shape=[
            jax.ShapeDtypeStruct((N, D), jnp.float32),
            jax.ShapeDtypeStruct((N, D), jnp.float32),
        ],
    )(P, base1, b1r, w2relr, w2rootr)


def _tc_out(Q, b2p, b2r):
    def body(q_ref, b_ref, b2_ref, o_ref):
        s = q_ref[0] + q_ref[1] + b_ref[...]
        o_ref[...] = jnp.sum(s, axis=1, keepdims=True) + b2_ref[0, 0]

    return pl.pallas_call(
        body,
        grid=(N // _BR,),
        in_specs=[
            pl.BlockSpec((NC, _BR, D), lambda i: (0, i, 0)),
            pl.BlockSpec((_BR, D), lambda i: (i, 0)),
            pl.BlockSpec((1, 1), lambda i: (0, 0)),
        ],
        out_specs=pl.BlockSpec((_BR, 1), lambda i: (i, 0)),
        out_shape=jax.ShapeDtypeStruct((N, 1), jnp.float32),
    )(Q, b2p, b2r)


def kernel(x, edge_index, W1_rel, b1, W1_root, W2_rel, b2, W2_root):
    src3 = edge_index[0].reshape(NW, NCHUNK, CHUNK)
    dst3 = edge_index[1].reshape(NW, NCHUNK, CHUNK)

    Wc = jnp.concatenate([W1_rel, W1_root], axis=1)
    y1, base1 = _tc_in(x, Wc)

    P = _segsum16(y1, src3, dst3)

    y2p, b2p = _tc_mid(
        P,
        base1,
        b1.reshape(1, D),
        W2_rel.reshape(1, D),
        W2_root.reshape(1, D),
    )

    Q = _segsum16(y2p, src3, dst3)

    return _tc_out(Q, b2p, b2.reshape(1, 1))


# speedup vs baseline: 18.6365x; 1.7490x over previous
import functools

import jax
import jax.numpy as jnp
from jax import lax
from jax.experimental import pallas as pl
from jax.experimental.pallas import tpu as pltpu
from jax.experimental.pallas import tpu_sc as plsc

N = 10000
D_IN = 128
D = 16
E = 320000
NC = 2
NS = 16
NW = NC * NS
E_PER_W = E // NW
CHUNK = 80
NCHUNK = E_PER_W // CHUNK
NP = 10240
ROWS_PER_TILE = NP // NS
ZROWS = 128


def _segsum16(y, src3, dst3):
    mesh = plsc.VectorSubcoreMesh(core_axis_name="c", subcore_axis_name="s")

    @functools.partial(
        pl.kernel,
        out_type=jax.ShapeDtypeStruct((NC, NP, D), jnp.float32),
        mesh=mesh,
        compiler_params=pltpu.CompilerParams(use_tc_tiling_on_sc=False),
        scratch_types=[
            pltpu.VMEM((NCHUNK, CHUNK), jnp.int32),
            pltpu.VMEM((NCHUNK, CHUNK), jnp.int32),
            pltpu.VMEM((CHUNK, D), jnp.float32),
            pltpu.VMEM((ZROWS, D), jnp.float32),
            pltpu.VMEM_SHARED((NP, D), jnp.float32),
            pltpu.VMEM_SHARED((NP, D), jnp.float32),
        ],
    )
    def k(y_hbm, src_hbm, dst_hbm, out_hbm, src_v, dst_v, rows_v, zbuf, ysp, acc):
        cid = lax.axis_index("c")
        sid = lax.axis_index("s")
        wid = cid * NS + sid

        pltpu.sync_copy(
            y_hbm.at[pl.ds(sid * ROWS_PER_TILE, ROWS_PER_TILE)],
            ysp.at[pl.ds(sid * ROWS_PER_TILE, ROWS_PER_TILE)],
        )

        zero16 = jnp.zeros((D,), jnp.float32)

        def zfill(i, _):
            zbuf[i] = zero16
            return 0

        lax.fori_loop(0, ZROWS, zfill, 0)

        def zcopy(i, _):
            pltpu.sync_copy(zbuf, acc.at[pl.ds(sid * ROWS_PER_TILE + i * ZROWS, ZROWS)])
            return 0

        lax.fori_loop(0, ROWS_PER_TILE // ZROWS, zcopy, 0)

        pltpu.sync_copy(src_hbm.at[wid], src_v)
        pltpu.sync_copy(dst_hbm.at[wid], dst_v)
        plsc.subcore_barrier()

        def body(j, _):
            pltpu.sync_copy(ysp.at[src_v.at[j]], rows_v)
            pltpu.sync_copy(rows_v, acc.at[dst_v.at[j]], add=True)
            return 0

        lax.fori_loop(0, NCHUNK, body, 0)
        plsc.subcore_barrier()

        pltpu.sync_copy(
            acc.at[pl.ds(sid * ROWS_PER_TILE, ROWS_PER_TILE)],
            out_hbm.at[cid, pl.ds(sid * ROWS_PER_TILE, ROWS_PER_TILE)],
        )

    return k(y, src3, dst3)


_BR = 1024


def _tc_in(x, Wc):
    def body(x_ref, w_ref, y_ref, b_ref):
        z = jnp.dot(x_ref[...], w_ref[...], preferred_element_type=jnp.float32)
        y_ref[...] = z[:, :D]
        b_ref[...] = z[:, D:]

    return pl.pallas_call(
        body,
        grid=(NP // _BR,),
        in_specs=[
            pl.BlockSpec((_BR, D_IN), lambda i: (i, 0)),
            pl.BlockSpec((D_IN, 2 * D), lambda i: (0, 0)),
        ],
        out_specs=[
            pl.BlockSpec((_BR, D), lambda i: (i, 0)),
            pl.BlockSpec((_BR, D), lambda i: (i, 0)),
        ],
        out_shape=[
            jax.ShapeDtypeStruct((NP, D), jnp.float32),
            jax.ShapeDtypeStruct((NP, D), jnp.float32),
        ],
    )(x, Wc)


def _tc_mid(P, base1, b1r, w2relr, w2rootr):
    def body(p_ref, base_ref, b1_ref, wr_ref, wo_ref, y_ref, o_ref):
        h = jnp.maximum(p_ref[0] + p_ref[1] + base_ref[...] + b1_ref[...], 0.0)
        y_ref[...] = h * wr_ref[...]
        o_ref[...] = h * wo_ref[...]

    return pl.pallas_call(
        body,
        grid=(NP // _BR,),
        in_specs=[
            pl.BlockSpec((NC, _BR, D), lambda i: (0, i, 0)),
            pl.BlockSpec((_BR, D), lambda i: (i, 0)),
            pl.BlockSpec((1, D), lambda i: (0, 0)),
            pl.BlockSpec((1, D), lambda i: (0, 0)),
            pl.BlockSpec((1, D), lambda i: (0, 0)),
        ],
        out_specs=[
            pl.BlockSpec((_BR, D), lambda i: (i, 0)),
            pl.BlockSpec((_BR, D), lambda i: (i, 0)),
        ],
        out_shape=[
            jax.ShapeDtypeStruct((NP, D), jnp.float32),
            jax.ShapeDtypeStruct((NP, D), jnp.float32),
        ],
    )(P, base1, b1r, w2relr, w2rootr)


_BO = 1000


def _tc_out(Q, b2p, b2r):
    def body(q_ref, b_ref, b2_ref, o_ref):
        s = q_ref[0] + q_ref[1] + b_ref[...]
        o_ref[...] = jnp.sum(s, axis=1, keepdims=True) + b2_ref[0, 0]

    return pl.pallas_call(
        body,
        grid=(N // _BO,),
        in_specs=[
            pl.BlockSpec((NC, _BO, D), lambda i: (0, i, 0)),
            pl.BlockSpec((_BO, D), lambda i: (i, 0)),
            pl.BlockSpec((1, 1), lambda i: (0, 0)),
        ],
        out_specs=pl.BlockSpec((_BO, 1), lambda i: (i, 0)),
        out_shape=jax.ShapeDtypeStruct((N, 1), jnp.float32),
    )(Q, b2p, b2r)


def kernel(x, edge_index, W1_rel, b1, W1_root, W2_rel, b2, W2_root):
    src3 = edge_index[0].reshape(NW, NCHUNK, CHUNK)
    dst3 = edge_index[1].reshape(NW, NCHUNK, CHUNK)

    xp = jnp.concatenate([x, jnp.zeros((NP - N, D_IN), x.dtype)], axis=0)
    Wc = jnp.concatenate([W1_rel, W1_root], axis=1)
    y1, base1 = _tc_in(xp, Wc)

    P = _segsum16(y1, src3, dst3)

    y2p, b2p = _tc_mid(
        P,
        base1,
        b1.reshape(1, D),
        W2_rel.reshape(1, D),
        W2_root.reshape(1, D),
    )

    Q = _segsum16(y2p, src3, dst3)

    return _tc_out(Q, b2p, b2.reshape(1, 1))


# speedup vs baseline: 18.6636x; 1.0015x over previous
import functools

import jax
import jax.numpy as jnp
from jax import lax
from jax.experimental import pallas as pl
from jax.experimental.pallas import tpu as pltpu
from jax.experimental.pallas import tpu_sc as plsc

N = 10000
D_IN = 128
D = 16
E = 320000
NC = 2
NS = 16
NW = NC * NS
CHUNK = 80
NCHUNK = 125
EP = NW * NCHUNK * CHUNK
NP = 10240
ROWS_PER_TILE = NP // NS
ZROWS = 128
NG = NCHUNK // 2


def _segsum16(y, src3, dst3):
    mesh = plsc.VectorSubcoreMesh(core_axis_name="c", subcore_axis_name="s")

    @functools.partial(
        pl.kernel,
        out_type=jax.ShapeDtypeStruct((NC, NP, D), jnp.float32),
        mesh=mesh,
        compiler_params=pltpu.CompilerParams(use_tc_tiling_on_sc=False),
        scratch_types=[
            pltpu.VMEM((NCHUNK, CHUNK), jnp.int32),
            pltpu.VMEM((NCHUNK, CHUNK), jnp.int32),
            pltpu.VMEM((CHUNK, D), jnp.float32),
            pltpu.VMEM((CHUNK, D), jnp.float32),
            pltpu.VMEM((ZROWS, D), jnp.float32),
            pltpu.VMEM_SHARED((NP, D), jnp.float32),
            pltpu.VMEM_SHARED((NP, D), jnp.float32),
            pltpu.SemaphoreType.DMA,
            pltpu.SemaphoreType.DMA,
            pltpu.SemaphoreType.DMA,
            pltpu.SemaphoreType.DMA,
        ],
    )
    def k(y_hbm, src_hbm, dst_hbm, out_hbm, src_v, dst_v, rows0, rows1,
          zbuf, ysp, acc, gsem0, gsem1, ssem0, ssem1):
        cid = lax.axis_index("c")
        sid = lax.axis_index("s")
        wid = cid * NS + sid

        pltpu.sync_copy(
            y_hbm.at[pl.ds(sid * ROWS_PER_TILE, ROWS_PER_TILE)],
            ysp.at[pl.ds(sid * ROWS_PER_TILE, ROWS_PER_TILE)],
        )

        zero16 = jnp.zeros((D,), jnp.float32)

        def zfill(i, _):
            zbuf[i] = zero16
            return 0

        lax.fori_loop(0, ZROWS, zfill, 0)

        def zcopy(i, _):
            pltpu.sync_copy(zbuf, acc.at[pl.ds(sid * ROWS_PER_TILE + i * ZROWS, ZROWS)])
            return 0

        lax.fori_loop(0, ROWS_PER_TILE // ZROWS, zcopy, 0)

        pltpu.sync_copy(src_hbm.at[wid], src_v)
        pltpu.sync_copy(dst_hbm.at[wid], dst_v)
        plsc.subcore_barrier()

        def body(j, _):
            pltpu.sync_copy(ysp.at[src_v.at[j]], rows0)
            pltpu.sync_copy(rows0, acc.at[dst_v.at[j]], add=True)
            return 0

        lax.fori_loop(0, NCHUNK, body, 0)
        plsc.subcore_barrier()

        pltpu.sync_copy(
            acc.at[pl.ds(sid * ROWS_PER_TILE, ROWS_PER_TILE)],
            out_hbm.at[cid, pl.ds(sid * ROWS_PER_TILE, ROWS_PER_TILE)],
        )

    return k(y, src3, dst3)


_BR = 1024


def _tc_in(x, Wc):
    def body(x_ref, w_ref, y_ref, b_ref):
        z = jnp.dot(x_ref[...], w_ref[...], preferred_element_type=jnp.float32)
        y_ref[...] = z[:, :D]
        b_ref[...] = z[:, D:]

    return pl.pallas_call(
        body,
        grid=(NP // _BR,),
        in_specs=[
            pl.BlockSpec((_BR, D_IN), lambda i: (i, 0)),
            pl.BlockSpec((D_IN, 2 * D), lambda i: (0, 0)),
        ],
        out_specs=[
            pl.BlockSpec((_BR, D), lambda i: (i, 0)),
            pl.BlockSpec((_BR, D), lambda i: (i, 0)),
        ],
        out_shape=[
            jax.ShapeDtypeStruct((NP, D), jnp.float32),
            jax.ShapeDtypeStruct((NP, D), jnp.float32),
        ],
    )(x, Wc)


def _tc_mid(P, base1, b1r, w2relr, w2rootr):
    def body(p_ref, base_ref, b1_ref, wr_ref, wo_ref, y_ref, o_ref):
        h = jnp.maximum(p_ref[0] + p_ref[1] + base_ref[...] + b1_ref[...], 0.0)
        y_ref[...] = h * wr_ref[...]
        o_ref[...] = h * wo_ref[...]

    return pl.pallas_call(
        body,
        grid=(NP // _BR,),
        in_specs=[
            pl.BlockSpec((NC, _BR, D), lambda i: (0, i, 0)),
            pl.BlockSpec((_BR, D), lambda i: (i, 0)),
            pl.BlockSpec((1, D), lambda i: (0, 0)),
            pl.BlockSpec((1, D), lambda i: (0, 0)),
            pl.BlockSpec((1, D), lambda i: (0, 0)),
        ],
        out_specs=[
            pl.BlockSpec((_BR, D), lambda i: (i, 0)),
            pl.BlockSpec((_BR, D), lambda i: (i, 0)),
        ],
        out_shape=[
            jax.ShapeDtypeStruct((NP, D), jnp.float32),
            jax.ShapeDtypeStruct((NP, D), jnp.float32),
        ],
    )(P, base1, b1r, w2relr, w2rootr)


_BO = 1000


def _tc_out(Q, b2p, b2r):
    def body(q_ref, b_ref, b2_ref, o_ref):
        s = q_ref[0] + q_ref[1] + b_ref[...]
        o_ref[...] = jnp.sum(s, axis=1, keepdims=True) + b2_ref[0, 0]

    return pl.pallas_call(
        body,
        grid=(N // _BO,),
        in_specs=[
            pl.BlockSpec((NC, _BO, D), lambda i: (0, i, 0)),
            pl.BlockSpec((_BO, D), lambda i: (i, 0)),
            pl.BlockSpec((1, 1), lambda i: (0, 0)),
        ],
        out_specs=pl.BlockSpec((_BO, 1), lambda i: (i, 0)),
        out_shape=jax.ShapeDtypeStruct((N, 1), jnp.float32),
    )(Q, b2p, b2r)


def kernel(x, edge_index, W1_rel, b1, W1_root, W2_rel, b2, W2_root):
    pad = jnp.full((EP - E,), N, jnp.int32)
    src3 = jnp.concatenate([edge_index[0], pad]).reshape(NW, NCHUNK, CHUNK)
    dst3 = jnp.concatenate([edge_index[1], pad]).reshape(NW, NCHUNK, CHUNK)

    xp = jnp.concatenate([x, jnp.zeros((NP - N, D_IN), x.dtype)], axis=0)
    Wc = jnp.concatenate([W1_rel, W1_root], axis=1)
    y1, base1 = _tc_in(xp, Wc)

    P = _segsum16(y1, src3, dst3)

    y2p, b2p = _tc_mid(
        P,
        base1,
        b1.reshape(1, D),
        W2_rel.reshape(1, D),
        W2_root.reshape(1, D),
    )

    Q = _segsum16(y2p, src3, dst3)

    return _tc_out(Q, b2p, b2.reshape(1, 1))


# speedup vs baseline: 21.3191x; 1.1423x over previous
import functools

import jax
import jax.numpy as jnp
from jax import lax
from jax.experimental import pallas as pl
from jax.experimental.pallas import tpu as pltpu
from jax.experimental.pallas import tpu_sc as plsc

N = 10000
D_IN = 128
D = 16
E = 320000
NC = 2
NS = 16
NW = NC * NS
CHUNK = 80
NCHUNK = 125
EP = NW * NCHUNK * CHUNK
NP = 10240
ROWS_PER_TILE = NP // NS
ZROWS = 128
NG = NCHUNK // 2


def _segsum16(y, src3, dst3):
    mesh = plsc.VectorSubcoreMesh(core_axis_name="c", subcore_axis_name="s")

    @functools.partial(
        pl.kernel,
        out_type=jax.ShapeDtypeStruct((NC, NP, D), jnp.float32),
        mesh=mesh,
        compiler_params=pltpu.CompilerParams(use_tc_tiling_on_sc=False),
        scratch_types=[
            pltpu.VMEM((NCHUNK, CHUNK), jnp.int32),
            pltpu.VMEM((NCHUNK, CHUNK), jnp.int32),
            pltpu.VMEM((CHUNK, D), jnp.float32),
            pltpu.VMEM((CHUNK, D), jnp.float32),
            pltpu.VMEM((ZROWS, D), jnp.float32),
            pltpu.VMEM_SHARED((NP, D), jnp.float32),
            pltpu.VMEM_SHARED((NP, D), jnp.float32),
            pltpu.SemaphoreType.DMA,
            pltpu.SemaphoreType.DMA,
            pltpu.SemaphoreType.DMA,
            pltpu.SemaphoreType.DMA,
        ],
    )
    def k(y_hbm, src_hbm, dst_hbm, out_hbm, src_v, dst_v, rows0, rows1,
          zbuf, ysp, acc, gsem0, gsem1, ssem0, ssem1):
        cid = lax.axis_index("c")
        sid = lax.axis_index("s")
        wid = cid * NS + sid

        pltpu.sync_copy(
            y_hbm.at[pl.ds(sid * ROWS_PER_TILE, ROWS_PER_TILE)],
            ysp.at[pl.ds(sid * ROWS_PER_TILE, ROWS_PER_TILE)],
        )

        zero16 = jnp.zeros((D,), jnp.float32)

        def zfill(i, _):
            zbuf[i] = zero16
            return 0

        lax.fori_loop(0, ZROWS, zfill, 0)

        def zcopy(i, _):
            pltpu.sync_copy(zbuf, acc.at[pl.ds(sid * ROWS_PER_TILE + i * ZROWS, ZROWS)])
            return 0

        lax.fori_loop(0, ROWS_PER_TILE // ZROWS, zcopy, 0)

        pltpu.sync_copy(src_hbm.at[wid], src_v)
        pltpu.sync_copy(dst_hbm.at[wid], dst_v)
        plsc.subcore_barrier()

        pltpu.async_copy(ysp.at[src_v.at[0]], rows0, gsem0)
        pltpu.async_copy(ysp.at[src_v.at[1]], rows1, gsem1)

        def body(g, _):
            j0 = 2 * g
            j1 = j0 + 1
            pltpu.make_async_copy(ysp.at[src_v.at[j0]], rows0, gsem0).wait()
            pltpu.async_copy(rows0, acc.at[dst_v.at[j0]], ssem0, add=True)
            pltpu.make_async_copy(ysp.at[src_v.at[j1]], rows1, gsem1).wait()
            pltpu.async_copy(rows1, acc.at[dst_v.at[j1]], ssem1, add=True)
            pltpu.make_async_copy(rows0, acc.at[dst_v.at[j0]], ssem0).wait()
            pltpu.async_copy(ysp.at[src_v.at[j0 + 2]], rows0, gsem0)
            pltpu.make_async_copy(rows1, acc.at[dst_v.at[j1]], ssem1).wait()

            @pl.when(j1 + 2 < NCHUNK)
            def _():
                pltpu.async_copy(ysp.at[src_v.at[j1 + 2]], rows1, gsem1)

            return 0

        lax.fori_loop(0, NG, body, 0)

        jl = NCHUNK - 1
        pltpu.make_async_copy(ysp.at[src_v.at[jl]], rows0, gsem0).wait()
        pltpu.sync_copy(rows0, acc.at[dst_v.at[jl]], add=True)
        plsc.subcore_barrier()

        pltpu.sync_copy(
            acc.at[pl.ds(sid * ROWS_PER_TILE, ROWS_PER_TILE)],
            out_hbm.at[cid, pl.ds(sid * ROWS_PER_TILE, ROWS_PER_TILE)],
        )

    return k(y, src3, dst3)


_BR = 1024


def _tc_in(x, Wc):
    def body(x_ref, w_ref, y_ref, b_ref):
        z = jnp.dot(x_ref[...], w_ref[...], preferred_element_type=jnp.float32)
        y_ref[...] = z[:, :D]
        b_ref[...] = z[:, D:]

    return pl.pallas_call(
        body,
        grid=(NP // _BR,),
        in_specs=[
            pl.BlockSpec((_BR, D_IN), lambda i: (i, 0)),
            pl.BlockSpec((D_IN, 2 * D), lambda i: (0, 0)),
        ],
        out_specs=[
            pl.BlockSpec((_BR, D), lambda i: (i, 0)),
            pl.BlockSpec((_BR, D), lambda i: (i, 0)),
        ],
        out_shape=[
            jax.ShapeDtypeStruct((NP, D), jnp.float32),
            jax.ShapeDtypeStruct((NP, D), jnp.float32),
        ],
    )(x, Wc)


def _tc_mid(P, base1, b1r, w2relr, w2rootr):
    def body(p_ref, base_ref, b1_ref, wr_ref, wo_ref, y_ref, o_ref):
        h = jnp.maximum(p_ref[0] + p_ref[1] + base_ref[...] + b1_ref[...], 0.0)
        y_ref[...] = h * wr_ref[...]
        o_ref[...] = h * wo_ref[...]

    return pl.pallas_call(
        body,
        grid=(NP // _BR,),
        in_specs=[
            pl.BlockSpec((NC, _BR, D), lambda i: (0, i, 0)),
            pl.BlockSpec((_BR, D), lambda i: (i, 0)),
            pl.BlockSpec((1, D), lambda i: (0, 0)),
            pl.BlockSpec((1, D), lambda i: (0, 0)),
            pl.BlockSpec((1, D), lambda i: (0, 0)),
        ],
        out_specs=[
            pl.BlockSpec((_BR, D), lambda i: (i, 0)),
            pl.BlockSpec((_BR, D), lambda i: (i, 0)),
        ],
        out_shape=[
            jax.ShapeDtypeStruct((NP, D), jnp.float32),
            jax.ShapeDtypeStruct((NP, D), jnp.float32),
        ],
    )(P, base1, b1r, w2relr, w2rootr)


_BO = 1000


def _tc_out(Q, b2p, b2r):
    def body(q_ref, b_ref, b2_ref, o_ref):
        s = q_ref[0] + q_ref[1] + b_ref[...]
        o_ref[...] = jnp.sum(s, axis=1, keepdims=True) + b2_ref[0, 0]

    return pl.pallas_call(
        body,
        grid=(N // _BO,),
        in_specs=[
            pl.BlockSpec((NC, _BO, D), lambda i: (0, i, 0)),
            pl.BlockSpec((_BO, D), lambda i: (i, 0)),
            pl.BlockSpec((1, 1), lambda i: (0, 0)),
        ],
        out_specs=pl.BlockSpec((_BO, 1), lambda i: (i, 0)),
        out_shape=jax.ShapeDtypeStruct((N, 1), jnp.float32),
    )(Q, b2p, b2r)


def kernel(x, edge_index, W1_rel, b1, W1_root, W2_rel, b2, W2_root):
    pad = jnp.full((EP - E,), N, jnp.int32)
    src3 = jnp.concatenate([edge_index[0], pad]).reshape(NW, NCHUNK, CHUNK)
    dst3 = jnp.concatenate([edge_index[1], pad]).reshape(NW, NCHUNK, CHUNK)

    xp = jnp.concatenate([x, jnp.zeros((NP - N, D_IN), x.dtype)], axis=0)
    Wc = jnp.concatenate([W1_rel, W1_root], axis=1)
    y1, base1 = _tc_in(xp, Wc)

    P = _segsum16(y1, src3, dst3)

    y2p, b2p = _tc_mid(
        P,
        base1,
        b1.reshape(1, D),
        W2_rel.reshape(1, D),
        W2_root.reshape(1, D),
    )

    Q = _segsum16(y2p, src3, dst3)

    return _tc_out(Q, b2p, b2.reshape(1, 1))


# speedup vs baseline: 22.7201x; 1.0657x over previous
import functools

import jax
import jax.numpy as jnp
from jax import lax
from jax.experimental import pallas as pl
from jax.experimental.pallas import tpu as pltpu
from jax.experimental.pallas import tpu_sc as plsc

N = 10000
D_IN = 128
D = 16
E = 320000
NC = 2
NS = 16
NW = NC * NS
CHUNK = 400
NCHUNK = 25
EP = NW * NCHUNK * CHUNK
NP = 10240
ROWS_PER_TILE = NP // NS
ZROWS = 128
NG = NCHUNK // 2


def _segsum16(y, src3, dst3):
    mesh = plsc.VectorSubcoreMesh(core_axis_name="c", subcore_axis_name="s")

    @functools.partial(
        pl.kernel,
        out_type=jax.ShapeDtypeStruct((NC, NP, D), jnp.float32),
        mesh=mesh,
        compiler_params=pltpu.CompilerParams(use_tc_tiling_on_sc=False),
        scratch_types=[
            pltpu.VMEM((NCHUNK, CHUNK), jnp.int32),
            pltpu.VMEM((NCHUNK, CHUNK), jnp.int32),
            pltpu.VMEM((CHUNK, D), jnp.float32),
            pltpu.VMEM((CHUNK, D), jnp.float32),
            pltpu.VMEM((ZROWS, D), jnp.float32),
            pltpu.VMEM_SHARED((NP, D), jnp.float32),
            pltpu.VMEM_SHARED((NP, D), jnp.float32),
            pltpu.SemaphoreType.DMA,
            pltpu.SemaphoreType.DMA,
            pltpu.SemaphoreType.DMA,
            pltpu.SemaphoreType.DMA,
        ],
    )
    def k(y_hbm, src_hbm, dst_hbm, out_hbm, src_v, dst_v, rows0, rows1,
          zbuf, ysp, acc, gsem0, gsem1, ssem0, ssem1):
        cid = lax.axis_index("c")
        sid = lax.axis_index("s")
        wid = cid * NS + sid

        pltpu.sync_copy(
            y_hbm.at[pl.ds(sid * ROWS_PER_TILE, ROWS_PER_TILE)],
            ysp.at[pl.ds(sid * ROWS_PER_TILE, ROWS_PER_TILE)],
        )

        zero16 = jnp.zeros((D,), jnp.float32)

        def zfill(i, _):
            zbuf[i] = zero16
            return 0

        lax.fori_loop(0, ZROWS, zfill, 0)

        def zcopy(i, _):
            pltpu.sync_copy(zbuf, acc.at[pl.ds(sid * ROWS_PER_TILE + i * ZROWS, ZROWS)])
            return 0

        lax.fori_loop(0, ROWS_PER_TILE // ZROWS, zcopy, 0)

        pltpu.sync_copy(src_hbm.at[wid], src_v)
        pltpu.sync_copy(dst_hbm.at[wid], dst_v)
        plsc.subcore_barrier()

        pltpu.async_copy(ysp.at[src_v.at[0]], rows0, gsem0)
        pltpu.async_copy(ysp.at[src_v.at[1]], rows1, gsem1)

        def body(g, _):
            j0 = 2 * g
            j1 = j0 + 1
            pltpu.make_async_copy(ysp.at[src_v.at[j0]], rows0, gsem0).wait()
            pltpu.async_copy(rows0, acc.at[dst_v.at[j0]], ssem0, add=True)
            pltpu.make_async_copy(ysp.at[src_v.at[j1]], rows1, gsem1).wait()
            pltpu.async_copy(rows1, acc.at[dst_v.at[j1]], ssem1, add=True)
            pltpu.make_async_copy(rows0, acc.at[dst_v.at[j0]], ssem0).wait()
            pltpu.async_copy(ysp.at[src_v.at[j0 + 2]], rows0, gsem0)
            pltpu.make_async_copy(rows1, acc.at[dst_v.at[j1]], ssem1).wait()

            @pl.when(j1 + 2 < NCHUNK)
            def _():
                pltpu.async_copy(ysp.at[src_v.at[j1 + 2]], rows1, gsem1)

            return 0

        lax.fori_loop(0, NG, body, 0)

        jl = NCHUNK - 1
        pltpu.make_async_copy(ysp.at[src_v.at[jl]], rows0, gsem0).wait()
        pltpu.sync_copy(rows0, acc.at[dst_v.at[jl]], add=True)
        plsc.subcore_barrier()

        pltpu.sync_copy(
            acc.at[pl.ds(sid * ROWS_PER_TILE, ROWS_PER_TILE)],
            out_hbm.at[cid, pl.ds(sid * ROWS_PER_TILE, ROWS_PER_TILE)],
        )

    return k(y, src3, dst3)


_BR = 1024


def _tc_in(x, Wc):
    def body(x_ref, w_ref, y_ref, b_ref):
        z = jnp.dot(x_ref[...], w_ref[...], preferred_element_type=jnp.float32)
        y_ref[...] = z[:, :D]
        b_ref[...] = z[:, D:]

    return pl.pallas_call(
        body,
        grid=(NP // _BR,),
        in_specs=[
            pl.BlockSpec((_BR, D_IN), lambda i: (i, 0)),
            pl.BlockSpec((D_IN, 2 * D), lambda i: (0, 0)),
        ],
        out_specs=[
            pl.BlockSpec((_BR, D), lambda i: (i, 0)),
            pl.BlockSpec((_BR, D), lambda i: (i, 0)),
        ],
        out_shape=[
            jax.ShapeDtypeStruct((NP, D), jnp.float32),
            jax.ShapeDtypeStruct((NP, D), jnp.float32),
        ],
    )(x, Wc)


def _tc_mid(P, base1, b1r, w2relr, w2rootr):
    def body(p_ref, base_ref, b1_ref, wr_ref, wo_ref, y_ref, o_ref):
        h = jnp.maximum(p_ref[0] + p_ref[1] + base_ref[...] + b1_ref[...], 0.0)
        y_ref[...] = h * wr_ref[...]
        o_ref[...] = h * wo_ref[...]

    return pl.pallas_call(
        body,
        grid=(NP // _BR,),
        in_specs=[
            pl.BlockSpec((NC, _BR, D), lambda i: (0, i, 0)),
            pl.BlockSpec((_BR, D), lambda i: (i, 0)),
            pl.BlockSpec((1, D), lambda i: (0, 0)),
            pl.BlockSpec((1, D), lambda i: (0, 0)),
            pl.BlockSpec((1, D), lambda i: (0, 0)),
        ],
        out_specs=[
            pl.BlockSpec((_BR, D), lambda i: (i, 0)),
            pl.BlockSpec((_BR, D), lambda i: (i, 0)),
        ],
        out_shape=[
            jax.ShapeDtypeStruct((NP, D), jnp.float32),
            jax.ShapeDtypeStruct((NP, D), jnp.float32),
        ],
    )(P, base1, b1r, w2relr, w2rootr)


_BO = 1000


def _tc_out(Q, b2p, b2r):
    def body(q_ref, b_ref, b2_ref, o_ref):
        s = q_ref[0] + q_ref[1] + b_ref[...]
        o_ref[...] = jnp.sum(s, axis=1, keepdims=True) + b2_ref[0, 0]

    return pl.pallas_call(
        body,
        grid=(N // _BO,),
        in_specs=[
            pl.BlockSpec((NC, _BO, D), lambda i: (0, i, 0)),
            pl.BlockSpec((_BO, D), lambda i: (i, 0)),
            pl.BlockSpec((1, 1), lambda i: (0, 0)),
        ],
        out_specs=pl.BlockSpec((_BO, 1), lambda i: (i, 0)),
        out_shape=jax.ShapeDtypeStruct((N, 1), jnp.float32),
    )(Q, b2p, b2r)


def kernel(x, edge_index, W1_rel, b1, W1_root, W2_rel, b2, W2_root):
    pad = jnp.full((EP - E,), N, jnp.int32)
    src3 = jnp.concatenate([edge_index[0], pad]).reshape(NW, NCHUNK, CHUNK)
    dst3 = jnp.concatenate([edge_index[1], pad]).reshape(NW, NCHUNK, CHUNK)

    xp = jnp.concatenate([x, jnp.zeros((NP - N, D_IN), x.dtype)], axis=0)
    Wc = jnp.concatenate([W1_rel, W1_root], axis=1)
    y1, base1 = _tc_in(xp, Wc)

    P = _segsum16(y1, src3, dst3)

    y2p, b2p = _tc_mid(
        P,
        base1,
        b1.reshape(1, D),
        W2_rel.reshape(1, D),
        W2_root.reshape(1, D),
    )

    Q = _segsum16(y2p, src3, dst3)

    return _tc_out(Q, b2p, b2.reshape(1, 1))


# speedup vs baseline: 24.0756x; 1.0597x over previous
import functools

import jax
import jax.numpy as jnp
from jax import lax
from jax.experimental import pallas as pl
from jax.experimental.pallas import tpu as pltpu
from jax.experimental.pallas import tpu_sc as plsc

N = 10000
D_IN = 128
D = 16
E = 320000
NC = 2
NS = 16
NW = NC * NS
CHUNK = 2000
NCHUNK = 5
EP = NW * NCHUNK * CHUNK
NP = 10240
ROWS_PER_TILE = NP // NS
ZROWS = 128
NG = NCHUNK // 2


def _segsum16(y, src3, dst3):
    mesh = plsc.VectorSubcoreMesh(core_axis_name="c", subcore_axis_name="s")

    @functools.partial(
        pl.kernel,
        out_type=jax.ShapeDtypeStruct((NC, NP, D), jnp.float32),
        mesh=mesh,
        compiler_params=pltpu.CompilerParams(use_tc_tiling_on_sc=False),
        scratch_types=[
            pltpu.VMEM((NCHUNK, CHUNK), jnp.int32),
            pltpu.VMEM((NCHUNK, CHUNK), jnp.int32),
            pltpu.VMEM((CHUNK, D), jnp.float32),
            pltpu.VMEM((CHUNK, D), jnp.float32),
            pltpu.VMEM((ZROWS, D), jnp.float32),
            pltpu.VMEM_SHARED((NP, D), jnp.float32),
            pltpu.VMEM_SHARED((NP, D), jnp.float32),
            pltpu.SemaphoreType.DMA,
            pltpu.SemaphoreType.DMA,
            pltpu.SemaphoreType.DMA,
            pltpu.SemaphoreType.DMA,
        ],
    )
    def k(y_hbm, src_hbm, dst_hbm, out_hbm, src_v, dst_v, rows0, rows1,
          zbuf, ysp, acc, gsem0, gsem1, ssem0, ssem1):
        cid = lax.axis_index("c")
        sid = lax.axis_index("s")
        wid = cid * NS + sid

        pltpu.sync_copy(
            y_hbm.at[pl.ds(sid * ROWS_PER_TILE, ROWS_PER_TILE)],
            ysp.at[pl.ds(sid * ROWS_PER_TILE, ROWS_PER_TILE)],
        )

        zero16 = jnp.zeros((D,), jnp.float32)

        def zfill(i, _):
            zbuf[i] = zero16
            return 0

        lax.fori_loop(0, ZROWS, zfill, 0)

        def zcopy(i, _):
            pltpu.sync_copy(zbuf, acc.at[pl.ds(sid * ROWS_PER_TILE + i * ZROWS, ZROWS)])
            return 0

        lax.fori_loop(0, ROWS_PER_TILE // ZROWS, zcopy, 0)

        pltpu.sync_copy(src_hbm.at[wid], src_v)
        pltpu.sync_copy(dst_hbm.at[wid], dst_v)
        plsc.subcore_barrier()

        pltpu.async_copy(ysp.at[src_v.at[0]], rows0, gsem0)
        pltpu.async_copy(ysp.at[src_v.at[1]], rows1, gsem1)

        def body(g, _):
            j0 = 2 * g
            j1 = j0 + 1
            pltpu.make_async_copy(ysp.at[src_v.at[j0]], rows0, gsem0).wait()
            pltpu.async_copy(rows0, acc.at[dst_v.at[j0]], ssem0, add=True)
            pltpu.make_async_copy(ysp.at[src_v.at[j1]], rows1, gsem1).wait()
            pltpu.async_copy(rows1, acc.at[dst_v.at[j1]], ssem1, add=True)
            pltpu.make_async_copy(rows0, acc.at[dst_v.at[j0]], ssem0).wait()
            pltpu.async_copy(ysp.at[src_v.at[j0 + 2]], rows0, gsem0)
            pltpu.make_async_copy(rows1, acc.at[dst_v.at[j1]], ssem1).wait()

            @pl.when(j1 + 2 < NCHUNK)
            def _():
                pltpu.async_copy(ysp.at[src_v.at[j1 + 2]], rows1, gsem1)

            return 0

        lax.fori_loop(0, NG, body, 0)

        jl = NCHUNK - 1
        pltpu.make_async_copy(ysp.at[src_v.at[jl]], rows0, gsem0).wait()
        pltpu.sync_copy(rows0, acc.at[dst_v.at[jl]], add=True)
        plsc.subcore_barrier()

        pltpu.sync_copy(
            acc.at[pl.ds(sid * ROWS_PER_TILE, ROWS_PER_TILE)],
            out_hbm.at[cid, pl.ds(sid * ROWS_PER_TILE, ROWS_PER_TILE)],
        )

    return k(y, src3, dst3)


_BR = 1024


def _tc_in(x, Wc):
    def body(x_ref, w_ref, y_ref, b_ref):
        z = jnp.dot(x_ref[...], w_ref[...], preferred_element_type=jnp.float32)
        y_ref[...] = z[:, :D]
        b_ref[...] = z[:, D:]

    return pl.pallas_call(
        body,
        grid=(NP // _BR,),
        in_specs=[
            pl.BlockSpec((_BR, D_IN), lambda i: (i, 0)),
            pl.BlockSpec((D_IN, 2 * D), lambda i: (0, 0)),
        ],
        out_specs=[
            pl.BlockSpec((_BR, D), lambda i: (i, 0)),
            pl.BlockSpec((_BR, D), lambda i: (i, 0)),
        ],
        out_shape=[
            jax.ShapeDtypeStruct((NP, D), jnp.float32),
            jax.ShapeDtypeStruct((NP, D), jnp.float32),
        ],
    )(x, Wc)


def _tc_mid(P, base1, b1r, w2relr, w2rootr):
    def body(p_ref, base_ref, b1_ref, wr_ref, wo_ref, y_ref, o_ref):
        h = jnp.maximum(p_ref[0] + p_ref[1] + base_ref[...] + b1_ref[...], 0.0)
        y_ref[...] = h * wr_ref[...]
        o_ref[...] = h * wo_ref[...]

    return pl.pallas_call(
        body,
        grid=(NP // _BR,),
        in_specs=[
            pl.BlockSpec((NC, _BR, D), lambda i: (0, i, 0)),
            pl.BlockSpec((_BR, D), lambda i: (i, 0)),
            pl.BlockSpec((1, D), lambda i: (0, 0)),
            pl.BlockSpec((1, D), lambda i: (0, 0)),
            pl.BlockSpec((1, D), lambda i: (0, 0)),
        ],
        out_specs=[
            pl.BlockSpec((_BR, D), lambda i: (i, 0)),
            pl.BlockSpec((_BR, D), lambda i: (i, 0)),
        ],
        out_shape=[
            jax.ShapeDtypeStruct((NP, D), jnp.float32),
            jax.ShapeDtypeStruct((NP, D), jnp.float32),
        ],
    )(P, base1, b1r, w2relr, w2rootr)


_BO = 1000


def _tc_out(Q, b2p, b2r):
    def body(q_ref, b_ref, b2_ref, o_ref):
        s = q_ref[0] + q_ref[1] + b_ref[...]
        o_ref[...] = jnp.sum(s, axis=1, keepdims=True) + b2_ref[0, 0]

    return pl.pallas_call(
        body,
        grid=(N // _BO,),
        in_specs=[
            pl.BlockSpec((NC, _BO, D), lambda i: (0, i, 0)),
            pl.BlockSpec((_BO, D), lambda i: (i, 0)),
            pl.BlockSpec((1, 1), lambda i: (0, 0)),
        ],
        out_specs=pl.BlockSpec((_BO, 1), lambda i: (i, 0)),
        out_shape=jax.ShapeDtypeStruct((N, 1), jnp.float32),
    )(Q, b2p, b2r)


def kernel(x, edge_index, W1_rel, b1, W1_root, W2_rel, b2, W2_root):
    pad = jnp.full((EP - E,), N, jnp.int32)
    src3 = jnp.concatenate([edge_index[0], pad]).reshape(NW, NCHUNK, CHUNK)
    dst3 = jnp.concatenate([edge_index[1], pad]).reshape(NW, NCHUNK, CHUNK)

    xp = jnp.concatenate([x, jnp.zeros((NP - N, D_IN), x.dtype)], axis=0)
    Wc = jnp.concatenate([W1_rel, W1_root], axis=1)
    y1, base1 = _tc_in(xp, Wc)

    P = _segsum16(y1, src3, dst3)

    y2p, b2p = _tc_mid(
        P,
        base1,
        b1.reshape(1, D),
        W2_rel.reshape(1, D),
        W2_root.reshape(1, D),
    )

    Q = _segsum16(y2p, src3, dst3)

    return _tc_out(Q, b2p, b2.reshape(1, 1))


# speedup vs baseline: 25.0873x; 1.0420x over previous
import functools

import jax
import jax.numpy as jnp
from jax import lax
from jax.experimental import pallas as pl
from jax.experimental.pallas import tpu as pltpu
from jax.experimental.pallas import tpu_sc as plsc

N = 10000
D_IN = 128
D = 16
E = 320000
NC = 2
NS = 16
NW = NC * NS
CHUNK = 2000
NCHUNK = 5
EP = NW * NCHUNK * CHUNK
NP = 10240
ROWS_PER_TILE = NP // NS
ZROWS = 128
NG = NCHUNK // 2


def _segsum16(y, src3, dst3):
    mesh = plsc.VectorSubcoreMesh(core_axis_name="c", subcore_axis_name="s")

    @functools.partial(
        pl.kernel,
        out_type=jax.ShapeDtypeStruct((NC, NP, D), jnp.float32),
        mesh=mesh,
        compiler_params=pltpu.CompilerParams(use_tc_tiling_on_sc=False),
        scratch_types=[
            pltpu.VMEM((NCHUNK, CHUNK), jnp.int32),
            pltpu.VMEM((NCHUNK, CHUNK), jnp.int32),
            pltpu.VMEM((CHUNK, D), jnp.float32),
            pltpu.VMEM((CHUNK, D), jnp.float32),
            pltpu.VMEM((ZROWS, D), jnp.float32),
            pltpu.VMEM_SHARED((NP, D), jnp.float32),
            pltpu.VMEM_SHARED((NP, D), jnp.float32),
            pltpu.SemaphoreType.DMA,
            pltpu.SemaphoreType.DMA,
            pltpu.SemaphoreType.DMA,
            pltpu.SemaphoreType.DMA,
        ],
    )
    def k(y_hbm, src_hbm, dst_hbm, out_hbm, src_v, dst_v, rows0, rows1,
          zbuf, ysp, acc, gsem0, gsem1, ssem0, ssem1):
        cid = lax.axis_index("c")
        sid = lax.axis_index("s")
        wid = cid * NS + sid

        pltpu.async_copy(
            y_hbm.at[pl.ds(sid * ROWS_PER_TILE, ROWS_PER_TILE)],
            ysp.at[pl.ds(sid * ROWS_PER_TILE, ROWS_PER_TILE)],
            gsem0,
        )
        pltpu.async_copy(src_hbm.at[wid], src_v, gsem1)
        pltpu.async_copy(dst_hbm.at[wid], dst_v, ssem0)

        zero16 = jnp.zeros((D,), jnp.float32)

        def zfill(i, _):
            zbuf[i] = zero16
            return 0

        lax.fori_loop(0, ZROWS, zfill, 0)

        for i in range(ROWS_PER_TILE // ZROWS):
            pltpu.async_copy(
                zbuf, acc.at[pl.ds(sid * ROWS_PER_TILE + i * ZROWS, ZROWS)], ssem1
            )

        pltpu.make_async_copy(
            y_hbm.at[pl.ds(sid * ROWS_PER_TILE, ROWS_PER_TILE)],
            ysp.at[pl.ds(sid * ROWS_PER_TILE, ROWS_PER_TILE)],
            gsem0,
        ).wait()
        pltpu.make_async_copy(src_hbm.at[wid], src_v, gsem1).wait()
        pltpu.make_async_copy(dst_hbm.at[wid], dst_v, ssem0).wait()
        for i in range(ROWS_PER_TILE // ZROWS):
            pltpu.make_async_copy(
                zbuf, acc.at[pl.ds(sid * ROWS_PER_TILE + i * ZROWS, ZROWS)], ssem1
            ).wait()
        plsc.subcore_barrier()

        pltpu.async_copy(ysp.at[src_v.at[0]], rows0, gsem0)
        pltpu.async_copy(ysp.at[src_v.at[1]], rows1, gsem1)

        def body(g, _):
            j0 = 2 * g
            j1 = j0 + 1
            pltpu.make_async_copy(ysp.at[src_v.at[j0]], rows0, gsem0).wait()
            pltpu.async_copy(rows0, acc.at[dst_v.at[j0]], ssem0, add=True)
            pltpu.make_async_copy(ysp.at[src_v.at[j1]], rows1, gsem1).wait()
            pltpu.async_copy(rows1, acc.at[dst_v.at[j1]], ssem1, add=True)
            pltpu.make_async_copy(rows0, acc.at[dst_v.at[j0]], ssem0).wait()
            pltpu.async_copy(ysp.at[src_v.at[j0 + 2]], rows0, gsem0)
            pltpu.make_async_copy(rows1, acc.at[dst_v.at[j1]], ssem1).wait()

            @pl.when(j1 + 2 < NCHUNK)
            def _():
                pltpu.async_copy(ysp.at[src_v.at[j1 + 2]], rows1, gsem1)

            return 0

        lax.fori_loop(0, NG, body, 0)

        jl = NCHUNK - 1
        pltpu.make_async_copy(ysp.at[src_v.at[jl]], rows0, gsem0).wait()
        pltpu.sync_copy(rows0, acc.at[dst_v.at[jl]], add=True)
        plsc.subcore_barrier()

        pltpu.sync_copy(
            acc.at[pl.ds(sid * ROWS_PER_TILE, ROWS_PER_TILE)],
            out_hbm.at[cid, pl.ds(sid * ROWS_PER_TILE, ROWS_PER_TILE)],
        )

    return k(y, src3, dst3)


_BR = 1024


def _tc_in(x, Wc):
    def body(x_ref, w_ref, y_ref, b_ref):
        z = jnp.dot(x_ref[...], w_ref[...], preferred_element_type=jnp.float32)
        y_ref[...] = z[:, :D]
        b_ref[...] = z[:, D:]

    return pl.pallas_call(
        body,
        grid=(NP // _BR,),
        in_specs=[
            pl.BlockSpec((_BR, D_IN), lambda i: (i, 0)),
            pl.BlockSpec((D_IN, 2 * D), lambda i: (0, 0)),
        ],
        out_specs=[
            pl.BlockSpec((_BR, D), lambda i: (i, 0)),
            pl.BlockSpec((_BR, D), lambda i: (i, 0)),
        ],
        out_shape=[
            jax.ShapeDtypeStruct((NP, D), jnp.float32),
            jax.ShapeDtypeStruct((NP, D), jnp.float32),
        ],
    )(x, Wc)


def _tc_mid(P, base1, b1r, w2relr, w2rootr):
    def body(p_ref, base_ref, b1_ref, wr_ref, wo_ref, y_ref, o_ref):
        h = jnp.maximum(p_ref[0] + p_ref[1] + base_ref[...] + b1_ref[...], 0.0)
        y_ref[...] = h * wr_ref[...]
        o_ref[...] = h * wo_ref[...]

    return pl.pallas_call(
        body,
        grid=(NP // _BR,),
        in_specs=[
            pl.BlockSpec((NC, _BR, D), lambda i: (0, i, 0)),
            pl.BlockSpec((_BR, D), lambda i: (i, 0)),
            pl.BlockSpec((1, D), lambda i: (0, 0)),
            pl.BlockSpec((1, D), lambda i: (0, 0)),
            pl.BlockSpec((1, D), lambda i: (0, 0)),
        ],
        out_specs=[
            pl.BlockSpec((_BR, D), lambda i: (i, 0)),
            pl.BlockSpec((_BR, D), lambda i: (i, 0)),
        ],
        out_shape=[
            jax.ShapeDtypeStruct((NP, D), jnp.float32),
            jax.ShapeDtypeStruct((NP, D), jnp.float32),
        ],
    )(P, base1, b1r, w2relr, w2rootr)


_BO = 1000


def _tc_out(Q, b2p, b2r):
    def body(q_ref, b_ref, b2_ref, o_ref):
        s = q_ref[0] + q_ref[1] + b_ref[...]
        o_ref[...] = jnp.sum(s, axis=1, keepdims=True) + b2_ref[0, 0]

    return pl.pallas_call(
        body,
        grid=(N // _BO,),
        in_specs=[
            pl.BlockSpec((NC, _BO, D), lambda i: (0, i, 0)),
            pl.BlockSpec((_BO, D), lambda i: (i, 0)),
            pl.BlockSpec((1, 1), lambda i: (0, 0)),
        ],
        out_specs=pl.BlockSpec((_BO, 1), lambda i: (i, 0)),
        out_shape=jax.ShapeDtypeStruct((N, 1), jnp.float32),
    )(Q, b2p, b2r)


def kernel(x, edge_index, W1_rel, b1, W1_root, W2_rel, b2, W2_root):
    pad = jnp.full((EP - E,), N, jnp.int32)
    src3 = jnp.concatenate([edge_index[0], pad]).reshape(NW, NCHUNK, CHUNK)
    dst3 = jnp.concatenate([edge_index[1], pad]).reshape(NW, NCHUNK, CHUNK)

    xp = jnp.concatenate([x, jnp.zeros((NP - N, D_IN), x.dtype)], axis=0)
    Wc = jnp.concatenate([W1_rel, W1_root], axis=1)
    y1, base1 = _tc_in(xp, Wc)

    P = _segsum16(y1, src3, dst3)

    y2p, b2p = _tc_mid(
        P,
        base1,
        b1.reshape(1, D),
        W2_rel.reshape(1, D),
        W2_root.reshape(1, D),
    )

    Q = _segsum16(y2p, src3, dst3)

    return _tc_out(Q, b2p, b2.reshape(1, 1))
